# Initial kernel scaffold; baseline (speedup 1.0000x reference)
import jax
import jax.numpy as jnp
from jax.experimental import pallas as pl


def kernel(filtre, X, node_indicator, W_gcn, b_gcn, W_cls, b_cls):
    raise NotImplementedError("write your pallas kernel here")


# speedup vs baseline: 3.5936x; 3.5936x over previous
import jax
import jax.numpy as jnp
from jax.experimental import pallas as pl
from jax.experimental.pallas import tpu as pltpu

N = 4096
D_IN = 32
D_GCN = 4
NUM_GRAPHS = 64
NUM_CLASSES = 10

BLK = 512


def _fused_kernel(ind_ref, filt_ref, x_ref, wg_ref, bg_ref, wc_ref, bc_ref,
                  out_ref, xw_ref, sacc_ref, cacc_ref):
    i = pl.program_id(0)
    nsteps = pl.num_programs(0)

    @pl.when(i == 0)
    def _init():
        sacc_ref[:] = jnp.zeros_like(sacc_ref)
        cacc_ref[:] = jnp.zeros_like(cacc_ref)
        xw = jnp.dot(x_ref[:], wg_ref[:], preferred_element_type=jnp.float32)
        xw_ref[:] = xw.astype(jnp.bfloat16)

    filt_b = filt_ref[:].astype(jnp.bfloat16)
    h = jnp.dot(filt_b, xw_ref[:], preferred_element_type=jnp.float32)
    h = jnp.maximum(h + bg_ref[:], 0.0)

    seg = ind_ref[:, pl.ds(i * BLK, BLK)]
    gid = jax.lax.broadcasted_iota(jnp.int32, (NUM_GRAPHS, BLK), 0)
    onehot_t = (gid == seg).astype(jnp.float32)
    sacc_ref[:] += jnp.dot(onehot_t, h, preferred_element_type=jnp.float32)
    cacc_ref[:] += jnp.sum(onehot_t, axis=1, keepdims=True)

    @pl.when(i == nsteps - 1)
    def _finish():
        pooled = sacc_ref[:] / jnp.maximum(cacc_ref[:], 1.0)
        logits = jnp.dot(pooled, wc_ref[:],
                         preferred_element_type=jnp.float32) + bc_ref[:]
        out_ref[:] = jax.nn.softmax(logits, axis=-1)


def kernel(filtre, X, node_indicator, W_gcn, b_gcn, W_cls, b_cls):
    ind = node_indicator.astype(jnp.int32).reshape(1, N)
    bg = b_gcn.reshape(1, D_GCN)
    bc = b_cls.reshape(1, NUM_CLASSES)
    nsteps = N // BLK

    return pl.pallas_call(
        _fused_kernel,
        grid=(nsteps,),
        in_specs=[
            pl.BlockSpec((1, N), lambda i: (0, 0)),
            pl.BlockSpec((BLK, N), lambda i: (i, 0)),
            pl.BlockSpec((N, D_IN), lambda i: (0, 0)),
            pl.BlockSpec((D_IN, D_GCN), lambda i: (0, 0)),
            pl.BlockSpec((1, D_GCN), lambda i: (0, 0)),
            pl.BlockSpec((D_GCN, NUM_CLASSES), lambda i: (0, 0)),
            pl.BlockSpec((1, NUM_CLASSES), lambda i: (0, 0)),
        ],
        out_specs=pl.BlockSpec((NUM_GRAPHS, NUM_CLASSES), lambda i: (0, 0)),
        scratch_shapes=[
            pltpu.VMEM((N, D_GCN), jnp.bfloat16),
            pltpu.VMEM((NUM_GRAPHS, D_GCN), jnp.float32),
            pltpu.VMEM((NUM_GRAPHS, 1), jnp.float32),
        ],
        out_shape=jax.ShapeDtypeStruct((NUM_GRAPHS, NUM_CLASSES),
                                       jnp.float32),
    )(ind, filtre, X, W_gcn, bg, W_cls, bc)
